# baseline (speedup 1.0000x reference)
import functools

import jax
import jax.numpy as jnp
from jax import lax
from jax.experimental import pallas as pl
from jax.experimental.pallas import tpu as pltpu
from jax.experimental.pallas import tpu_sc as plsc

_NUM_EXPERTS = 16
_TOP_K = 2
_N_TOKENS = 16384
_S = _N_TOKENS * _TOP_K
_LANES = 16
_NW = 32
_PER_TILE = _S // _NW
_PER_LANE = _PER_TILE // _LANES
_PER_SC_SLICE = _S // _LANES

_SC_PARAMS = pltpu.CompilerParams(needs_layout_passes=False)


def _mesh():
    return plsc.VectorSubcoreMesh(core_axis_name="c", subcore_axis_name="s")


def _hist_body(x_ref, tb_ref, cnt_ref, rs_ref):
    x = x_ref[...]
    for e in range(_NUM_EXPERTS):
        m = (x == e).astype(jnp.float32)
        rs_ref[:, e:e + 1] = jnp.sum(m, axis=1, keepdims=True)
    rs = rs_ref[...]

    rw = lax.broadcasted_iota(jnp.int32, (_NW, 256), 0)
    cr = lax.broadcasted_iota(jnp.int32, (_NW, 256), 1)
    wmat = (lax.shift_right_logical(cr, 3) == rw).astype(jnp.float32)
    ht = jnp.dot(wmat, rs, preferred_element_type=jnp.float32)

    total = jnp.sum(ht, axis=0, keepdims=True)
    cnt_ref[...] = total

    ru = lax.broadcasted_iota(jnp.int32, (_NUM_EXPERTS, _NUM_EXPERTS), 0)
    cu = lax.broadcasted_iota(jnp.int32, (_NUM_EXPERTS, _NUM_EXPERTS), 1)
    upper = (ru < cu).astype(jnp.float32)
    g = jnp.dot(total, upper, preferred_element_type=jnp.float32,
                precision=lax.Precision.HIGHEST)

    rl = lax.broadcasted_iota(jnp.int32, (_NW, _NW), 0)
    cl = lax.broadcasted_iota(jnp.int32, (_NW, _NW), 1)
    lower = (cl < rl).astype(jnp.float32)
    tb = jnp.dot(lower, ht, preferred_element_type=jnp.float32,
                 precision=lax.Precision.HIGHEST) + g
    tb_ref[:, 0:_NUM_EXPERTS] = tb


_hist = pl.pallas_call(
    _hist_body,
    out_shape=(
        jax.ShapeDtypeStruct((_NW, 128), jnp.float32),
        jax.ShapeDtypeStruct((1, _NUM_EXPERTS), jnp.float32),
    ),
    scratch_shapes=[pltpu.VMEM((256, _NUM_EXPERTS), jnp.float32)],
)


@functools.partial(
    pl.kernel,
    out_type=(
        jax.ShapeDtypeStruct((_S,), jnp.float32),
        jax.ShapeDtypeStruct((_S,), jnp.int32),
        jax.ShapeDtypeStruct((_S,), jnp.float32),
        jax.ShapeDtypeStruct((_S,), jnp.int32),
    ),
    mesh=_mesh(),
    compiler_params=_SC_PARAMS,
    scratch_types=[
        pltpu.VMEM((_PER_TILE,), jnp.int32),
        pltpu.VMEM((_PER_TILE,), jnp.float32),
        pltpu.VMEM((_PER_TILE,), jnp.int32),
        pltpu.VMEM((8, 128), jnp.int32),
        pltpu.VMEM((_LANES * _NUM_EXPERTS,), jnp.int32),
        pltpu.VMEM((_NUM_EXPERTS,), jnp.float32),
        pltpu.VMEM((_PER_SC_SLICE // 4,), jnp.float32),
        pltpu.VMEM((_PER_SC_SLICE // 4,), jnp.int32),
        pltpu.VMEM_SHARED((_S,), jnp.float32),
        pltpu.VMEM_SHARED((_S,), jnp.int32),
        pltpu.SemaphoreType.DMA,
        pltpu.SemaphoreType.DMA,
        pltpu.SemaphoreType.DMA,
        pltpu.SemaphoreType.DMA,
    ],
)
def _reorder(sel_hbm, sc_hbm, tb_hbm,
             pa_sc, pa_tok, pb_sc, pb_tok,
             sel_v, sc_v, tok_v, pos_v, cnt_v, tbrow_v,
             zf_v, zi_v, sp_sc, sp_tok, sem, sem_z, sem_in, sem_sc):
    c = lax.axis_index("c")
    s = lax.axis_index("s")
    w = c * _LANES + s
    base = w * _PER_TILE

    lane = jnp.arange(_LANES, dtype=jnp.int32)
    lane16 = lane * _NUM_EXPERTS
    zf = jnp.zeros((_LANES,), jnp.float32)
    zi = jnp.zeros((_LANES,), jnp.int32)

    d_sel = pltpu.async_copy(sel_hbm.at[pl.ds(base, _PER_TILE)], sel_v, sem_in)
    d_tb = pltpu.async_copy(tb_hbm.at[pl.ds(w * 128, _NUM_EXPERTS)],
                            tbrow_v, sem_in)
    d_sc = pltpu.async_copy(sc_hbm.at[pl.ds(base, _PER_TILE)], sc_v, sem_sc)

    _ZCH = _PER_SC_SLICE // 4

    def zero_body(j, carry):
        zf_v[pl.ds(j * _LANES, _LANES)] = zf
        zi_v[pl.ds(j * _LANES, _LANES)] = zi
        return carry

    lax.fori_loop(0, _ZCH // _LANES, zero_body, 0)
    zdescs = []
    for q in range(4):
        zsl = pl.ds(s * _PER_SC_SLICE + q * _ZCH, _ZCH)
        zdescs.append(pltpu.async_copy(zf_v, sp_sc.at[zsl], sem_z))
        zdescs.append(pltpu.async_copy(zi_v, sp_tok.at[zsl], sem_z))
    sl = pl.ds(s * _PER_SC_SLICE, _PER_SC_SLICE)

    d_sel.wait()
    d_tb.wait()

    def addrmap(i):
        return (
            lax.bitwise_and(i, jnp.int32(~255))
            | lax.shift_left(lax.bitwise_and(i, 1), 7)
            | lax.shift_right_logical(lax.bitwise_and(i, 255), 1)
        )

    for l in range(_LANES):
        cnt_v[pl.ds(l * _NUM_EXPERTS, _NUM_EXPERTS)] = zi

    def hist_body(t, carry):
        ad = addrmap(lane * _PER_LANE + t)
        e = plsc.load_gather(sel_v, [ad])
        a = lane16 + e
        cc = plsc.load_gather(cnt_v, [a])
        plsc.store_scatter(cnt_v, [a], cc + 1)
        al = t * _LANES + lane
        inv = (
            lax.bitwise_and(al, jnp.int32(~255))
            | lax.shift_left(lax.bitwise_and(al, 127), 1)
            | lax.bitwise_and(lax.shift_right_logical(al, 7), 1)
        )
        tok_v[pl.ds(t * _LANES, _LANES)] = lax.shift_right_logical(
            base + inv, 1)
        return carry

    lax.fori_loop(0, _PER_LANE, hist_body, 0)

    run = tbrow_v[...].astype(jnp.int32)
    for l in range(_LANES):
        csl = pl.ds(l * _NUM_EXPERTS, _NUM_EXPERTS)
        hl = cnt_v[csl]
        cnt_v[csl] = run
        run = run + hl

    def body(u, carry):
        for d in range(2):
            t = u * 2 + d
            ad = addrmap(lane * _PER_LANE + t)
            e = plsc.load_gather(sel_v, [ad])
            a = lane16 + e
            cc = plsc.load_gather(cnt_v, [a])
            plsc.store_scatter(cnt_v, [a], cc + 1)
            plsc.store_scatter(
                pos_v,
                [lax.shift_right_logical(ad, 7), lax.bitwise_and(ad, 127)],
                cc,
            )
        return carry

    lax.fori_loop(0, _PER_LANE // 2, body, 0)

    for d in zdescs:
        d.wait()
    d_sc.wait()
    plsc.subcore_barrier()

    descs = []
    for j in range(8):
        pj = pos_v.at[j]
        descs.append(pltpu.async_copy(
            sc_v.at[pl.ds(j * 128, 128)], sp_sc.at[pj], sem, add=True))
        descs.append(pltpu.async_copy(
            tok_v.at[pl.ds(j * 128, 128)], sp_tok.at[pj], sem, add=True))
    for d in descs:
        d.wait()

    plsc.subcore_barrier()

    @pl.when(c == 0)
    def _():
        o1 = pltpu.async_copy(sp_sc.at[sl], pa_sc.at[sl], sem_in)
        o2 = pltpu.async_copy(sp_tok.at[sl], pa_tok.at[sl], sem_sc)
        o1.wait()
        o2.wait()

    @pl.when(c == 1)
    def _():
        o1 = pltpu.async_copy(sp_sc.at[sl], pb_sc.at[sl], sem_in)
        o2 = pltpu.async_copy(sp_tok.at[sl], pb_tok.at[sl], sem_sc)
        o1.wait()
        o2.wait()


def _merge_body(pa_sc, pb_sc, pa_tok, pb_tok, out_sc, out_tok):
    out_sc[...] = pa_sc[...] + pb_sc[...]
    out_tok[...] = pa_tok[...] + pb_tok[...]


_merge = pl.pallas_call(
    _merge_body,
    out_shape=(
        jax.ShapeDtypeStruct((_S // 128, 128), jnp.float32),
        jax.ShapeDtypeStruct((_S // 128, 128), jnp.int32),
    ),
)


def kernel(top_scores, selected_experts_indices):
    sel_lin = jnp.transpose(
        selected_experts_indices.astype(jnp.int32).reshape(128, 128, _TOP_K),
        (0, 2, 1)).reshape(_S)
    sc_lin = jnp.transpose(
        top_scores.reshape(128, 128, _TOP_K), (0, 2, 1)).reshape(_S)

    tb4, cnt = _hist(sel_lin.reshape(256, 128))
    tb_lin = tb4.reshape(-1)

    pa_sc, pa_tok, pb_sc, pb_tok = _reorder(sel_lin, sc_lin, tb_lin)

    out_sc, out_tok = _merge(
        pa_sc.reshape(_S // 128, 128), pb_sc.reshape(_S // 128, 128),
        pa_tok.reshape(_S // 128, 128), pb_tok.reshape(_S // 128, 128))

    return out_sc.reshape(-1), out_tok.reshape(-1), cnt.reshape(-1)


# speedup vs baseline: 1.0001x; 1.0001x over previous
import functools

import jax
import jax.numpy as jnp
from jax import lax
from jax.experimental import pallas as pl
from jax.experimental.pallas import tpu as pltpu
from jax.experimental.pallas import tpu_sc as plsc

_NUM_EXPERTS = 16
_TOP_K = 2
_N_TOKENS = 16384
_S = _N_TOKENS * _TOP_K
_LANES = 16
_NW = 32
_PER_TILE = _S // _NW
_PER_LANE = _PER_TILE // _LANES
_PER_SC_SLICE = _S // _LANES

_SC_PARAMS = pltpu.CompilerParams(needs_layout_passes=False)


def _mesh():
    return plsc.VectorSubcoreMesh(core_axis_name="c", subcore_axis_name="s")


def _hist_body(x_ref, tb_ref, cnt_ref, rs_ref):
    x = x_ref[...]
    rw = lax.broadcasted_iota(jnp.int32, (_NW, 256), 0)
    cr = lax.broadcasted_iota(jnp.int32, (_NW, 256), 1)
    wmat = (lax.shift_right_logical(cr, 3) == rw).astype(jnp.float32)
    for e in range(_NUM_EXPERTS):
        m = (x == e).astype(jnp.float32)
        rs = jnp.sum(m, axis=1, keepdims=True)
        rs_ref[:, e:e + 1] = jnp.dot(wmat, rs,
                                     preferred_element_type=jnp.float32)
    ht = rs_ref[...]

    total = jnp.sum(ht, axis=0, keepdims=True)
    cnt_ref[...] = total

    ru = lax.broadcasted_iota(jnp.int32, (_NUM_EXPERTS, _NUM_EXPERTS), 0)
    cu = lax.broadcasted_iota(jnp.int32, (_NUM_EXPERTS, _NUM_EXPERTS), 1)
    upper = (ru < cu).astype(jnp.float32)
    g = jnp.dot(total, upper, preferred_element_type=jnp.float32,
                precision=lax.Precision.HIGHEST)

    rl = lax.broadcasted_iota(jnp.int32, (_NW, _NW), 0)
    cl = lax.broadcasted_iota(jnp.int32, (_NW, _NW), 1)
    lower = (cl < rl).astype(jnp.float32)
    tb = jnp.dot(lower, ht, preferred_element_type=jnp.float32,
                 precision=lax.Precision.HIGHEST) + g
    tb_ref[:, 0:_NUM_EXPERTS] = tb


_hist = pl.pallas_call(
    _hist_body,
    out_shape=(
        jax.ShapeDtypeStruct((_NW, 128), jnp.float32),
        jax.ShapeDtypeStruct((1, _NUM_EXPERTS), jnp.float32),
    ),
    scratch_shapes=[pltpu.VMEM((_NW, _NUM_EXPERTS), jnp.float32)],
)


@functools.partial(
    pl.kernel,
    out_type=(
        jax.ShapeDtypeStruct((_S,), jnp.float32),
        jax.ShapeDtypeStruct((_S,), jnp.int32),
        jax.ShapeDtypeStruct((_S,), jnp.float32),
        jax.ShapeDtypeStruct((_S,), jnp.int32),
    ),
    mesh=_mesh(),
    compiler_params=_SC_PARAMS,
    scratch_types=[
        pltpu.VMEM((_PER_TILE,), jnp.int32),
        pltpu.VMEM((_PER_TILE,), jnp.float32),
        pltpu.VMEM((_PER_TILE,), jnp.int32),
        pltpu.VMEM((8, 128), jnp.int32),
        pltpu.VMEM((_LANES * _NUM_EXPERTS,), jnp.int32),
        pltpu.VMEM((_NUM_EXPERTS,), jnp.float32),
        pltpu.VMEM((_PER_SC_SLICE // 4,), jnp.float32),
        pltpu.VMEM((_PER_SC_SLICE // 4,), jnp.int32),
        pltpu.VMEM_SHARED((_S,), jnp.float32),
        pltpu.VMEM_SHARED((_S,), jnp.int32),
        pltpu.SemaphoreType.DMA,
        pltpu.SemaphoreType.DMA,
        pltpu.SemaphoreType.DMA,
        pltpu.SemaphoreType.DMA,
    ],
)
def _reorder(sel_hbm, sc_hbm, tb_hbm,
             pa_sc, pa_tok, pb_sc, pb_tok,
             sel_v, sc_v, tok_v, pos_v, cnt_v, tbrow_v,
             zf_v, zi_v, sp_sc, sp_tok, sem, sem_z, sem_in, sem_sc):
    c = lax.axis_index("c")
    s = lax.axis_index("s")
    w = c * _LANES + s
    base = w * _PER_TILE

    lane = jnp.arange(_LANES, dtype=jnp.int32)
    lane16 = lane * _NUM_EXPERTS
    zf = jnp.zeros((_LANES,), jnp.float32)
    zi = jnp.zeros((_LANES,), jnp.int32)

    d_sel = pltpu.async_copy(sel_hbm.at[pl.ds(base, _PER_TILE)], sel_v, sem_in)
    d_tb = pltpu.async_copy(tb_hbm.at[pl.ds(w * 128, _NUM_EXPERTS)],
                            tbrow_v, sem_in)
    d_sc = pltpu.async_copy(sc_hbm.at[pl.ds(base, _PER_TILE)], sc_v, sem_sc)

    _ZCH = _PER_SC_SLICE // 4

    def zero_body(j, carry):
        zf_v[pl.ds(j * _LANES, _LANES)] = zf
        zi_v[pl.ds(j * _LANES, _LANES)] = zi
        return carry

    lax.fori_loop(0, _ZCH // _LANES, zero_body, 0)
    zdescs = []
    for q in range(4):
        zsl = pl.ds(s * _PER_SC_SLICE + q * _ZCH, _ZCH)
        zdescs.append(pltpu.async_copy(zf_v, sp_sc.at[zsl], sem_z))
        zdescs.append(pltpu.async_copy(zi_v, sp_tok.at[zsl], sem_z))
    sl = pl.ds(s * _PER_SC_SLICE, _PER_SC_SLICE)

    d_sel.wait()
    d_tb.wait()

    def addrmap(i):
        return (
            lax.bitwise_and(i, jnp.int32(~255))
            | lax.shift_left(lax.bitwise_and(i, 1), 7)
            | lax.shift_right_logical(lax.bitwise_and(i, 255), 1)
        )

    for l in range(_LANES):
        cnt_v[pl.ds(l * _NUM_EXPERTS, _NUM_EXPERTS)] = zi

    def hist_body(t, carry):
        ad = addrmap(lane * _PER_LANE + t)
        e = plsc.load_gather(sel_v, [ad])
        a = lane16 + e
        cc = plsc.load_gather(cnt_v, [a])
        plsc.store_scatter(cnt_v, [a], cc + 1)
        al = t * _LANES + lane
        inv = (
            lax.bitwise_and(al, jnp.int32(~255))
            | lax.shift_left(lax.bitwise_and(al, 127), 1)
            | lax.bitwise_and(lax.shift_right_logical(al, 7), 1)
        )
        tok_v[pl.ds(t * _LANES, _LANES)] = lax.shift_right_logical(
            base + inv, 1)
        return carry

    lax.fori_loop(0, _PER_LANE, hist_body, 0)

    run = tbrow_v[...].astype(jnp.int32)
    for l in range(_LANES):
        csl = pl.ds(l * _NUM_EXPERTS, _NUM_EXPERTS)
        hl = cnt_v[csl]
        cnt_v[csl] = run
        run = run + hl

    def body(u, carry):
        for d in range(2):
            t = u * 2 + d
            ad = addrmap(lane * _PER_LANE + t)
            e = plsc.load_gather(sel_v, [ad])
            a = lane16 + e
            cc = plsc.load_gather(cnt_v, [a])
            plsc.store_scatter(cnt_v, [a], cc + 1)
            plsc.store_scatter(
                pos_v,
                [lax.shift_right_logical(ad, 7), lax.bitwise_and(ad, 127)],
                cc,
            )
        return carry

    lax.fori_loop(0, _PER_LANE // 2, body, 0)

    for d in zdescs:
        d.wait()
    d_sc.wait()
    plsc.subcore_barrier()

    descs = []
    for j in range(8):
        pj = pos_v.at[j]
        descs.append(pltpu.async_copy(
            sc_v.at[pl.ds(j * 128, 128)], sp_sc.at[pj], sem, add=True))
        descs.append(pltpu.async_copy(
            tok_v.at[pl.ds(j * 128, 128)], sp_tok.at[pj], sem, add=True))
    for d in descs:
        d.wait()

    plsc.subcore_barrier()

    @pl.when(c == 0)
    def _():
        o1 = pltpu.async_copy(sp_sc.at[sl], pa_sc.at[sl], sem_in)
        o2 = pltpu.async_copy(sp_tok.at[sl], pa_tok.at[sl], sem_sc)
        o1.wait()
        o2.wait()

    @pl.when(c == 1)
    def _():
        o1 = pltpu.async_copy(sp_sc.at[sl], pb_sc.at[sl], sem_in)
        o2 = pltpu.async_copy(sp_tok.at[sl], pb_tok.at[sl], sem_sc)
        o1.wait()
        o2.wait()


def _merge_body(pa_sc, pb_sc, pa_tok, pb_tok, out_sc, out_tok):
    out_sc[...] = pa_sc[...] + pb_sc[...]
    out_tok[...] = pa_tok[...] + pb_tok[...]


_merge = pl.pallas_call(
    _merge_body,
    out_shape=(
        jax.ShapeDtypeStruct((_S // 128, 128), jnp.float32),
        jax.ShapeDtypeStruct((_S // 128, 128), jnp.int32),
    ),
)


def kernel(top_scores, selected_experts_indices):
    sel_lin = jnp.transpose(
        selected_experts_indices.astype(jnp.int32).reshape(128, 128, _TOP_K),
        (0, 2, 1)).reshape(_S)
    sc_lin = jnp.transpose(
        top_scores.reshape(128, 128, _TOP_K), (0, 2, 1)).reshape(_S)

    tb4, cnt = _hist(sel_lin.reshape(256, 128))
    tb_lin = tb4.reshape(-1)

    pa_sc, pa_tok, pb_sc, pb_tok = _reorder(sel_lin, sc_lin, tb_lin)

    out_sc, out_tok = _merge(
        pa_sc.reshape(_S // 128, 128), pb_sc.reshape(_S // 128, 128),
        pa_tok.reshape(_S // 128, 128), pb_tok.reshape(_S // 128, 128))

    return out_sc.reshape(-1), out_tok.reshape(-1), cnt.reshape(-1)
